# Initial kernel scaffold; baseline (speedup 1.0000x reference)
import jax
import jax.numpy as jnp
from jax.experimental import pallas as pl


def kernel(cost_features, hardware_features, w_cost, b_cost, g_cost, be_cost, w_hw, b_hw, g_hw, be_hw, in_proj_w, in_proj_b, out_proj_w, out_proj_b, w_fus, b_fus, g_fus, be_fus, w_out1, b_out1, w_out2, b_out2, w_unc1, b_unc1, w_unc2, b_unc2):
    raise NotImplementedError("write your pallas kernel here")


# speedup vs baseline: 16.4209x; 16.4209x over previous
import functools

import jax
import jax.numpy as jnp
import numpy as np
from jax.experimental import pallas as pl

E = 64
H = 64
NH = 8
HD = H // NH


def _ln(x, g, b):
    m = jnp.mean(x, axis=-1, keepdims=True)
    c = x - m
    v = jnp.mean(c * c, axis=-1, keepdims=True)
    return c * jax.lax.rsqrt(v + 1e-5) * g + b


def _gelu(x):
    return 0.5 * x * (1.0 + jax.lax.erf(x * np.float32(1.0 / np.sqrt(2.0))))


def _router_kernel(cf_ref, hf_ref,
                   wc_ref, bc_ref, gc_ref, bec_ref,
                   wh_ref, bh_ref, gh_ref, beh_ref,
                   wq_ref, wk_ref, wv_ref, bq_ref, bk_ref, bv_ref,
                   wo_ref, bo_ref,
                   wf_ref, bf_ref, gf_ref, bef_ref,
                   w1_ref, b1_ref, w2_ref, b2_ref,
                   wu1_ref, bu1_ref, wu2_ref, bu2_ref,
                   rb_ref, unc_ref):
    f32 = jnp.float32

    ce = _gelu(_ln(jnp.dot(cf_ref[...], wc_ref[...],
                           preferred_element_type=f32) + bc_ref[...],
                   gc_ref[...], bec_ref[...]))
    he = _gelu(_ln(jnp.dot(hf_ref[...], wh_ref[...],
                           preferred_element_type=f32) + bh_ref[...],
                   gh_ref[...], beh_ref[...]))

    bq = bq_ref[...]; bk = bk_ref[...]; bv = bv_ref[...]
    q_c = jnp.dot(ce, wq_ref[...], preferred_element_type=f32) + bq
    k_c = jnp.dot(ce, wk_ref[...], preferred_element_type=f32) + bk
    v_c = jnp.dot(ce, wv_ref[...], preferred_element_type=f32) + bv
    q_h = jnp.dot(he, wq_ref[...], preferred_element_type=f32) + bq
    k_h = jnp.dot(he, wk_ref[...], preferred_element_type=f32) + bk
    v_h = jnp.dot(he, wv_ref[...], preferred_element_type=f32) + bv

    row = jax.lax.broadcasted_iota(jnp.int32, (H, NH), 0)
    col = jax.lax.broadcasted_iota(jnp.int32, (H, NH), 1)
    sel = (row // HD == col).astype(f32)
    selT = sel.T

    scale = np.float32(1.0 / np.sqrt(HD))

    def head_dot(a, b):
        return jnp.dot(a * b, sel, preferred_element_type=f32) * scale

    s_cc = head_dot(q_c, k_c)
    s_ch = head_dot(q_c, k_h)
    s_ct = jnp.dot(q_c * bk, sel, preferred_element_type=f32) * scale
    s_hc = head_dot(q_h, k_c)
    s_hh = head_dot(q_h, k_h)
    s_ht = jnp.dot(q_h * bk, sel, preferred_element_type=f32) * scale
    s_tc = jnp.dot(bq * k_c, sel, preferred_element_type=f32) * scale
    s_th = jnp.dot(bq * k_h, sel, preferred_element_type=f32) * scale
    s_tt = jnp.dot(bq * bk, sel, preferred_element_type=f32) * scale

    def softmax3(a, b, c):
        m = jnp.maximum(jnp.maximum(a, b), c)
        ea = jnp.exp(a - m); eb = jnp.exp(b - m); ec = jnp.exp(c - m)
        inv = 1.0 / (ea + eb + ec)
        return ea * inv, eb * inv, ec * inv

    a_cc, a_ch, a_ct = softmax3(s_cc, s_ch, s_ct)
    a_hc, a_hh, a_ht = softmax3(s_hc, s_hh, s_ht)
    zt = jnp.zeros_like(s_tc)
    a_tc, a_th, a_tt = softmax3(s_tc, s_th, s_tt + zt)

    third = np.float32(1.0 / 3.0)
    w_vc = (a_cc + a_hc + a_tc) * third
    w_vh = (a_ch + a_hh + a_th) * third
    w_vt = (a_ct + a_ht + a_tt) * third

    o = (jnp.dot(w_vc, selT, preferred_element_type=f32) * v_c
         + jnp.dot(w_vh, selT, preferred_element_type=f32) * v_h
         + jnp.dot(w_vt, selT, preferred_element_type=f32) * bv)
    att_mean = jnp.dot(o, wo_ref[...], preferred_element_type=f32) + bo_ref[...]

    fused = _gelu(_ln(jnp.dot(att_mean, wf_ref[...],
                              preferred_element_type=f32) + bf_ref[...],
                      gf_ref[...], bef_ref[...]))

    h1 = _gelu(jnp.dot(fused, w1_ref[...], preferred_element_type=f32)
               + b1_ref[...])
    rb = jnp.tanh(jnp.dot(h1, w2_ref[...], preferred_element_type=f32)
                  + b2_ref[...])
    hu = _gelu(jnp.dot(fused, wu1_ref[...], preferred_element_type=f32)
               + bu1_ref[...])
    pre = jnp.dot(hu, wu2_ref[...], preferred_element_type=f32) + bu2_ref[...]
    unc = jnp.logaddexp(pre, 0.0)

    rb_ref[...] = rb
    unc_ref[...] = unc


@functools.partial(jax.jit, static_argnames=())
def kernel(cost_features, hardware_features, w_cost, b_cost, g_cost, be_cost,
           w_hw, b_hw, g_hw, be_hw, in_proj_w, in_proj_b, out_proj_w,
           out_proj_b, w_fus, b_fus, g_fus, be_fus, w_out1, b_out1, w_out2,
           b_out2, w_unc1, b_unc1, w_unc2, b_unc2):
    B, CD = cost_features.shape
    T = 1024
    grid = (B // T,)

    row2 = lambda v: v.reshape(1, -1)
    wq = in_proj_w[:H].T; wk = in_proj_w[H:2 * H].T; wv = in_proj_w[2 * H:].T
    bq = row2(in_proj_b[:H]); bk = row2(in_proj_b[H:2 * H])
    bv = row2(in_proj_b[2 * H:])

    full = lambda a: pl.BlockSpec(a.shape, lambda i: (0,) * a.ndim)
    operands = [
        cost_features, hardware_features,
        w_cost.T, row2(b_cost), row2(g_cost), row2(be_cost),
        w_hw.T, row2(b_hw), row2(g_hw), row2(be_hw),
        wq, wk, wv, bq, bk, bv,
        out_proj_w.T, row2(out_proj_b),
        w_fus.T, row2(b_fus), row2(g_fus), row2(be_fus),
        w_out1.T, row2(b_out1), w_out2.T, row2(b_out2),
        w_unc1.T, row2(b_unc1), w_unc2.T, row2(b_unc2),
    ]
    in_specs = [pl.BlockSpec((T, CD), lambda i: (i, 0)),
                pl.BlockSpec((T, 8), lambda i: (i, 0))]
    in_specs += [full(a) for a in operands[2:]]

    out_shape = [jax.ShapeDtypeStruct((B, E), jnp.float32),
                 jax.ShapeDtypeStruct((B, E), jnp.float32)]
    out_specs = [pl.BlockSpec((T, E), lambda i: (i, 0)),
                 pl.BlockSpec((T, E), lambda i: (i, 0))]

    rb, unc = pl.pallas_call(
        _router_kernel,
        grid=grid,
        in_specs=in_specs,
        out_specs=out_specs,
        out_shape=out_shape,
    )(*operands)
    return rb, unc
